# baseline (speedup 1.0000x reference)
import functools

import jax
import jax.numpy as jnp
from jax import lax
from jax.experimental import pallas as pl
from jax.experimental.pallas import tpu as pltpu
from jax.experimental.pallas import tpu_sc as plsc

_NUM_TABLES = 1000
_NUM_LAYER = 12
_BATCH = 4096
_D = _NUM_LAYER * 32 * 32
_NC, _NS = 2, 16
_NW = _NC * _NS
_K = 4
_SC_CALLS = 4


def _make_gather(batch):
    b_per_w = batch // _NW
    n_chunk = b_per_w // _K
    mesh = plsc.VectorSubcoreMesh(core_axis_name="c", subcore_axis_name="s")

    @functools.partial(
        pl.kernel,
        mesh=mesh,
        out_type=jax.ShapeDtypeStruct((batch, _D), jnp.float32),
        scratch_types=[
            pltpu.VMEM((n_chunk, _K), jnp.int32),
            pltpu.VMEM((_K, _D), jnp.float32),
            pltpu.VMEM((_K, _D), jnp.float32),
            pltpu.SemaphoreType.DMA,
            pltpu.SemaphoreType.DMA,
            pltpu.SemaphoreType.DMA,
            pltpu.SemaphoreType.DMA,
        ],
    )
    def gather_kernel(idx_hbm, table_hbm, out_hbm, idx_v,
                      buf0, buf1, gsem0, gsem1, osem0, osem1):
        wid = lax.axis_index("s") * _NC + lax.axis_index("c")
        pltpu.sync_copy(idx_hbm.at[wid], idx_v)
        base = wid * b_per_w
        bufs = (buf0, buf1)
        gsems = (gsem0, gsem1)
        osems = (osem0, osem1)

        def wait_gather(p):
            pltpu.make_async_copy(
                table_hbm.at[idx_v.at[0]], bufs[p], gsems[p]).wait()

        def wait_out(p):
            pltpu.make_async_copy(
                bufs[p], out_hbm.at[pl.ds(0, _K)], osems[p]).wait()

        def start_gather(j, p):
            pltpu.async_copy(table_hbm.at[idx_v.at[j]], bufs[p], gsems[p])

        def start_out(j, p):
            pltpu.async_copy(bufs[p], out_hbm.at[pl.ds(base + j * _K, _K)],
                             osems[p])

        start_gather(0, 0)
        wait_gather(0)
        start_out(0, 0)
        start_gather(1, 1)

        def body(i, carry):
            j0 = 2 * i + 1
            wait_gather(1)
            start_out(j0, 1)
            wait_out(0)
            start_gather(j0 + 1, 0)
            wait_gather(0)
            start_out(j0 + 1, 0)
            wait_out(1)
            start_gather(j0 + 2, 1)
            return carry

        lax.fori_loop(0, n_chunk // 2 - 1, body, 0)

        j_last = n_chunk - 1
        wait_gather(1)
        start_out(j_last, 1)
        wait_out(0)
        wait_out(1)

    return gather_kernel


_gather = _make_gather(_BATCH // _SC_CALLS)


def kernel(img_index, table):
    table2 = table.reshape(_NUM_TABLES, _D)
    bc = _BATCH // _SC_CALLS
    outs = []
    for c in range(_SC_CALLS):
        idx3 = lax.slice(img_index, (c * bc,), ((c + 1) * bc,)).reshape(
            _NW, bc // _NW // _K, _K)
        out2 = _gather(idx3, table2)
        outs.append(out2.reshape(bc, _NUM_LAYER, 32, 32))
    return jnp.concatenate(outs, axis=0)
